# Initial kernel scaffold; baseline (speedup 1.0000x reference)
import jax
import jax.numpy as jnp
from jax.experimental import pallas as pl


def kernel(marked_img, now_step):
    raise NotImplementedError("write your pallas kernel here")


# speedup vs baseline: 6.1499x; 6.1499x over previous
import functools

import jax
import jax.numpy as jnp
from jax import lax
from jax.experimental import pallas as pl
from jax.experimental.pallas import tpu as pltpu
from jax.experimental.pallas import tpu_sc as plsc

NOISE_RATIO = 0.1
NOISE_PROB = 0.5
MAX_STEP = 30

_H = 512
_W = 512
_P = _H * _W
_N = int(NOISE_RATIO * _P)
_NPAD = ((_N + 15) // 16) * 16

_NW = 32
_CH = _P // _NW
_LANES = 16

_sc_mesh = plsc.VectorSubcoreMesh(core_axis_name="c", subcore_axis_name="s")


@functools.partial(
    pl.kernel,
    mesh=_sc_mesh,
    out_type=jax.ShapeDtypeStruct((_P,), jnp.float32),
    scratch_types=[
        pltpu.VMEM((_NPAD,), jnp.int32),
        pltpu.VMEM((_NPAD,), jnp.float32),
        pltpu.VMEM((_CH,), jnp.float32),
    ],
    compiler_params=pltpu.CompilerParams(needs_layout_passes=False),
)
def _mask_build(idx_hbm, vals_hbm, out_hbm, idx_v, vals_v, buf):
    wid = lax.axis_index("s") * 2 + lax.axis_index("c")
    lo = wid * _CH
    pltpu.sync_copy(idx_hbm, idx_v)
    pltpu.sync_copy(vals_hbm, vals_v)

    ones = jnp.full((_LANES,), 1.0, jnp.float32)

    def init_body(i, carry):
        buf[pl.ds(i * _LANES, _LANES)] = ones
        return carry

    lax.fori_loop(0, _CH // _LANES, init_body, 0)

    lov = jnp.full((_LANES,), lo, jnp.int32)
    hiv = lov + _CH
    zero = jnp.zeros((_LANES,), jnp.int32)

    def scatter_body(i, carry):
        idx = idx_v[pl.ds(i * _LANES, _LANES)]
        v = vals_v[pl.ds(i * _LANES, _LANES)]
        m = (idx >= lov) & (idx < hiv)
        local = jnp.where(m, idx - lov, zero)
        plsc.store_scatter(buf, [local], v, mask=m)
        return carry

    lax.fori_loop(0, _NPAD // _LANES, scatter_body, 0)

    pltpu.sync_copy(buf, out_hbm.at[pl.ds(lo, _CH)])


def _tc_body(img_ref, mask_ref, out_ref):
    out_ref[...] = jnp.clip(img_ref[...] * mask_ref[...][None, :, :], 0.0, 1.0)


def kernel(marked_img, now_step):
    B, C, H, W = marked_img.shape
    num_noisy_pixels = _N

    key = jax.random.key(42)
    kp, kn = jax.random.split(key)
    indices = jax.random.permutation(kp, H * W)[:num_noisy_pixels]
    indices = indices.astype(jnp.int32)
    random_noise = jax.random.uniform(kn, (num_noisy_pixels,), dtype=jnp.float32)
    base_vals = jnp.where(random_noise < NOISE_PROB, 1.0, 0.0).astype(jnp.float32)

    noise_ratio_t = jnp.minimum(now_step / MAX_STEP, 1.0) * NOISE_RATIO
    num_noisy_pixels_t = noise_ratio_t * H * W
    vals = jnp.where(
        jnp.arange(num_noisy_pixels) < num_noisy_pixels_t, base_vals, 1.0
    ).astype(jnp.float32)

    pad = _NPAD - num_noisy_pixels
    idx_full = jnp.concatenate([indices, jnp.full((pad,), _P, jnp.int32)])
    vals_full = jnp.concatenate([vals, jnp.ones((pad,), jnp.float32)])

    mask_flat = _mask_build(idx_full, vals_full)
    mask2d = mask_flat.reshape(H, W)

    img3 = marked_img.reshape(B * C, H, W)
    out3 = pl.pallas_call(
        _tc_body,
        grid=(B * C,),
        in_specs=[
            pl.BlockSpec((1, H, W), lambda i: (i, 0, 0)),
            pl.BlockSpec((H, W), lambda i: (0, 0)),
        ],
        out_specs=pl.BlockSpec((1, H, W), lambda i: (i, 0, 0)),
        out_shape=jax.ShapeDtypeStruct((B * C, H, W), jnp.float32),
    )(img3, mask2d)
    return out3.reshape(B, C, H, W)


# speedup vs baseline: 6.5532x; 1.0656x over previous
import functools

import jax
import jax.numpy as jnp
from jax import lax
from jax.experimental import pallas as pl
from jax.experimental.pallas import tpu as pltpu
from jax.experimental.pallas import tpu_sc as plsc

NOISE_RATIO = 0.1
NOISE_PROB = 0.5
MAX_STEP = 30

_H = 512
_W = 512
_P = _H * _W
_N = int(NOISE_RATIO * _P)
_NPAD = ((_N + 15) // 16) * 16

_NW = 32
_CH = _P // _NW
_LANES = 16

_sc_mesh = plsc.VectorSubcoreMesh(core_axis_name="c", subcore_axis_name="s")


@functools.partial(
    pl.kernel,
    mesh=_sc_mesh,
    out_type=jax.ShapeDtypeStruct((_P,), jnp.float32),
    scratch_types=[
        pltpu.VMEM((_NPAD,), jnp.int32),
        pltpu.VMEM((_NPAD,), jnp.float32),
        pltpu.VMEM((_CH,), jnp.float32),
    ],
    compiler_params=pltpu.CompilerParams(needs_layout_passes=False),
)
def _mask_build(idx_hbm, vals_hbm, out_hbm, idx_v, vals_v, buf):
    wid = lax.axis_index("s") * 2 + lax.axis_index("c")
    lo = wid * _CH
    pltpu.sync_copy(idx_hbm, idx_v)
    pltpu.sync_copy(vals_hbm, vals_v)

    ones = jnp.full((_LANES,), 1.0, jnp.float32)

    def init_body(i, carry):
        buf[pl.ds(i * _LANES, _LANES)] = ones
        return carry

    lax.fori_loop(0, _CH // _LANES, init_body, 0)

    lov = jnp.full((_LANES,), lo, jnp.int32)
    hiv = lov + _CH
    zero = jnp.zeros((_LANES,), jnp.int32)

    def scatter_body(i, carry):
        idx = idx_v[pl.ds(i * _LANES, _LANES)]
        v = vals_v[pl.ds(i * _LANES, _LANES)]
        m = (idx >= lov) & (idx < hiv)
        local = jnp.where(m, idx - lov, zero)
        plsc.store_scatter(buf, [local], v, mask=m)
        return carry

    lax.fori_loop(0, _NPAD // _LANES, scatter_body, 0)

    pltpu.sync_copy(buf, out_hbm.at[pl.ds(lo, _CH)])


def _tc_body(img_ref, mask_ref, out_ref):
    out_ref[...] = jnp.clip(img_ref[...] * mask_ref[...][None, :, :], 0.0, 1.0)


def kernel(marked_img, now_step):
    B, C, H, W = marked_img.shape
    num_noisy_pixels = _N

    key = jax.random.key(42)
    kp, kn = jax.random.split(key)
    indices = jax.random.permutation(kp, H * W)[:num_noisy_pixels]
    indices = indices.astype(jnp.int32)
    random_noise = jax.random.uniform(kn, (num_noisy_pixels,), dtype=jnp.float32)
    base_vals = jnp.where(random_noise < NOISE_PROB, 1.0, 0.0).astype(jnp.float32)

    noise_ratio_t = jnp.minimum(now_step / MAX_STEP, 1.0) * NOISE_RATIO
    num_noisy_pixels_t = noise_ratio_t * H * W
    vals = jnp.where(
        jnp.arange(num_noisy_pixels) < num_noisy_pixels_t, base_vals, 1.0
    ).astype(jnp.float32)

    pad = _NPAD - num_noisy_pixels
    idx_full = jnp.concatenate([indices, jnp.full((pad,), _P, jnp.int32)])
    vals_full = jnp.concatenate([vals, jnp.ones((pad,), jnp.float32)])

    mask_flat = _mask_build(idx_full, vals_full)
    mask2d = mask_flat.reshape(H, W)

    img3 = marked_img.reshape(B * C, H, W)
    blk = 8
    out3 = pl.pallas_call(
        _tc_body,
        grid=(B * C // blk,),
        in_specs=[
            pl.BlockSpec((blk, H, W), lambda i: (i, 0, 0)),
            pl.BlockSpec((H, W), lambda i: (0, 0)),
        ],
        out_specs=pl.BlockSpec((blk, H, W), lambda i: (i, 0, 0)),
        out_shape=jax.ShapeDtypeStruct((B * C, H, W), jnp.float32),
    )(img3, mask2d)
    return out3.reshape(B, C, H, W)
